# baseline (speedup 1.0000x reference)
import jax, jax.numpy as jnp
from jax.experimental import pallas as pl


def _gatv2(x, src, dst, eattr, Wl, bl, Wr, br, We, att, b, n):
    xl = x @ Wl + bl
    xr = x @ Wr + br
    m = jax.nn.leaky_relu(xl[src] + xr[dst] + eattr @ We, 0.2)
    alpha = m @ att
    amax = jax.lax.stop_gradient(jax.ops.segment_max(alpha, dst, num_segments=n))
    amax = jnp.where(jnp.isfinite(amax), amax, 0.0)
    ex = jnp.exp(alpha - amax[dst])
    den = jax.ops.segment_sum(ex, dst, num_segments=n)
    a = ex / den[dst]
    return jax.ops.segment_sum(xl[src] * a[:, None], dst, num_segments=n) + b


def kernel(x, edge_index, edge_attr, batch, Wl1, bl1, Wr1, br1, We1, att1, b1, Wl2, bl2, Wr2, br2, We2, att2, b2, Wl3, bl3, Wr3, br3, We3, att3, b3, fc1_w, fc1_b, bn_g, bn_b, fc2_w, fc2_b):
    n = x.shape[0]
    src0, dst0 = edge_index[0], edge_index[1]
    cnt = jnp.maximum(jax.ops.segment_sum(jnp.ones((src0.shape[0],), jnp.float32), dst0, num_segments=n), 1.0)
    loop_attr = jax.ops.segment_sum(edge_attr, dst0, num_segments=n) / cnt[:, None]
    ar = jnp.arange(n, dtype=src0.dtype)
    src = jnp.concatenate([src0, ar])
    dst = jnp.concatenate([dst0, ar])
    ea = jnp.concatenate([edge_attr, loop_attr], axis=0)
    h = jax.nn.relu(_gatv2(x, src, dst, ea, Wl1, bl1, Wr1, br1, We1, att1, b1, n))
    h = jax.nn.relu(_gatv2(h, src, dst, ea, Wl2, bl2, Wr2, br2, We2, att2, b2, n))
    h = jax.nn.relu(_gatv2(h, src, dst, ea, Wl3, bl3, Wr3, br3, We3, att3, b3, n))
    gsum = jax.ops.segment_sum(h, batch, num_segments=64)
    gcnt = jnp.maximum(jax.ops.segment_sum(jnp.ones((n,), jnp.float32), batch, num_segments=64), 1.0)
    g = gsum / gcnt[:, None]
    z = g @ fc1_w + fc1_b
    mu = jnp.mean(z, axis=0)
    var = jnp.var(z, axis=0)
    z = (z - mu) / jnp.sqrt(var + 1e-5) * bn_g + bn_b
    z = jax.nn.relu(z)
    return z @ fc2_w + fc2_b
